# baseline (device time: 17570 ns/iter reference)
import jax
import jax.numpy as jnp
from jax import lax
from jax.experimental import pallas as pl
from jax.experimental.pallas import tpu as pltpu

N_DEV = 4
N_MSG = 12


def _gelu(y):
    c = 0.7978845608028654
    return 0.5 * y * (1.0 + jnp.tanh(c * (y + 0.044715 * y * y * y)))


def kernel(x, w_mat):
    m, k = x.shape
    _, n = w_mat.shape
    m_out = m // N_DEV
    nh = n // 2
    nq = n // 4

    def body(x_ref, w_ref, out_ref, send_buf, recv_buf, send_sems, recv_sems):
        d = lax.axis_index("i")
        left = lax.rem(d + N_DEV - 1, N_DEV)
        right = lax.rem(d + 1, N_DEV)

        def mm(c, lo, width):
            return jnp.dot(
                x_ref[pl.ds(lax.rem(c, N_DEV) * m_out, m_out), :],
                w_ref[:, lo:lo + width],
                preferred_element_type=jnp.float32,
            )

        barrier_sem = pltpu.get_barrier_semaphore()
        for nbr in (left, right):
            pl.semaphore_signal(
                barrier_sem, inc=1,
                device_id=(nbr,), device_id_type=pl.DeviceIdType.MESH,
            )
        pl.semaphore_wait(barrier_sem, 2)

        def make(j):
            return pltpu.make_async_remote_copy(
                src_ref=send_buf.at[j],
                dst_ref=recv_buf.at[j],
                send_sem=send_sems.at[j],
                recv_sem=recv_sems.at[j],
                device_id=(left if j < 6 else right,),
                device_id_type=pl.DeviceIdType.MESH,
            )

        sends = []

        def put(j, val):
            send_buf[j, :, :] = val.astype(jnp.bfloat16)
            s = make(j)
            s.start()
            sends.append(s)

        seed_a = mm(d + 2, 0, nh)
        put(0, seed_a[:, :nq])
        put(1, seed_a[:, nq:])
        seed_b = mm(d + 2, nh, nh)
        put(6, seed_b[:, :nq])
        put(7, seed_b[:, nq:])

        lm1 = mm(d + 3, 0, n)
        put(2, lm1[:, 2 * nq:3 * nq])
        put(3, lm1[:, 3 * nq:])
        lp1 = mm(d + 1, 0, n)
        put(8, lp1[:, :nq])
        put(9, lp1[:, nq:2 * nq])

        own = mm(d, 0, n)

        make(0).wait_recv()
        put(4, lm1[:, :nq] + recv_buf[0, :, :].astype(jnp.float32))
        make(6).wait_recv()
        put(10, lp1[:, 2 * nq:3 * nq] + recv_buf[6, :, :].astype(jnp.float32))
        make(1).wait_recv()
        put(5, lm1[:, nq:2 * nq] + recv_buf[1, :, :].astype(jnp.float32))
        make(7).wait_recv()
        put(11, lp1[:, 3 * nq:] + recv_buf[7, :, :].astype(jnp.float32))

        make(8).wait_recv()
        pre0 = own[:, :nq] + recv_buf[8, :, :].astype(jnp.float32)
        make(9).wait_recv()
        pre1 = own[:, nq:2 * nq] + recv_buf[9, :, :].astype(jnp.float32)
        make(2).wait_recv()
        pre2 = own[:, 2 * nq:3 * nq] + recv_buf[2, :, :].astype(jnp.float32)
        make(3).wait_recv()
        pre3 = own[:, 3 * nq:] + recv_buf[3, :, :].astype(jnp.float32)

        make(4).wait_recv()
        out_ref[:, :nq] = _gelu(pre0 + recv_buf[4, :, :].astype(jnp.float32))
        make(10).wait_recv()
        out_ref[:, 2 * nq:3 * nq] = _gelu(
            pre2 + recv_buf[10, :, :].astype(jnp.float32)
        )
        make(5).wait_recv()
        out_ref[:, nq:2 * nq] = _gelu(
            pre1 + recv_buf[5, :, :].astype(jnp.float32)
        )
        make(11).wait_recv()
        out_ref[:, 3 * nq:] = _gelu(
            pre3 + recv_buf[11, :, :].astype(jnp.float32)
        )

        for s in sends:
            s.wait_send()

    return pl.pallas_call(
        body,
        out_shape=jax.ShapeDtypeStruct((m_out, n), jnp.float32),
        in_specs=[
            pl.BlockSpec(memory_space=pltpu.VMEM),
            pl.BlockSpec(memory_space=pltpu.VMEM),
        ],
        out_specs=pl.BlockSpec(memory_space=pltpu.VMEM),
        scratch_shapes=[
            pltpu.VMEM((N_MSG, m_out, nq), jnp.bfloat16),
            pltpu.VMEM((N_MSG, m_out, nq), jnp.bfloat16),
            pltpu.SemaphoreType.DMA((N_MSG,)),
            pltpu.SemaphoreType.DMA((N_MSG,)),
        ],
        compiler_params=pltpu.CompilerParams(collective_id=0),
    )(x, w_mat)
